# baseline (speedup 1.0000x reference)
import functools

import jax
import jax.numpy as jnp
from jax import lax
from jax.experimental import pallas as pl
from jax.experimental.pallas import tpu as pltpu
from jax.experimental.pallas import tpu_sc as plsc

D_MODEL = 128
VOCAB = 100000

_NC = 2
_NS = 16
_NW = _NC * _NS
_NBUF = 10
_CH = 64


def _gather(table, idx_t, batch, n_ctx):
    per_w = batch // _NW
    mesh = plsc.VectorSubcoreMesh(core_axis_name="c", subcore_axis_name="s")

    @functools.partial(
        pl.kernel,
        mesh=mesh,
        out_type=jax.ShapeDtypeStruct((n_ctx, batch, D_MODEL), jnp.float32),
        scratch_types=(
            [pltpu.VMEM((n_ctx, per_w), jnp.int32)]
            + [pltpu.VMEM((_CH, D_MODEL), jnp.float32)] * _NBUF
            + [pltpu.SemaphoreType.DMA] * (2 * _NBUF)
        ),
    )
    def k(table_hbm, idx_hbm, out_hbm, idx_v, *bufs_and_sems):
        rows = bufs_and_sems[:_NBUF]
        gsems = bufs_and_sems[_NBUF:2 * _NBUF]
        wsems = bufs_and_sems[2 * _NBUF:]
        wid = lax.axis_index("s") * _NC + lax.axis_index("c")
        c0 = wid * per_w
        pltpu.sync_copy(idx_hbm.at[:, pl.ds(c0, per_w)], idx_v)

        def drain_write(i):
            pltpu.make_async_copy(
                rows[i], out_hbm.at[0, pl.ds(c0, _CH)], wsems[i]).wait()

        steps_per_col = per_w // _CH
        rounds = n_ctx * steps_per_col // _NBUF

        def body(r, carry):
            for i in range(_NBUF):
                s = r * _NBUF + i
                p, h = s // steps_per_col, s % steps_per_col

                @pl.when(r >= 1)
                def _(i=i):
                    drain_write(i)

                pltpu.async_copy(
                    table_hbm.at[idx_v.at[p, pl.ds(h * _CH, _CH)]], rows[i],
                    gsems[i])
            for i in range(_NBUF):
                s = r * _NBUF + i
                p, h = s // steps_per_col, s % steps_per_col
                pltpu.make_async_copy(
                    table_hbm.at[idx_v.at[p, pl.ds(h * _CH, _CH)]], rows[i],
                    gsems[i]).wait()
                pltpu.async_copy(
                    rows[i], out_hbm.at[p, pl.ds(c0 + h * _CH, _CH)],
                    wsems[i])
            return carry

        lax.fori_loop(0, rounds, body, 0)
        for i in range(_NBUF):
            drain_write(i)

    return k(table, idx_t)


def kernel(x, W_E):
    b, p = x.shape
    table = jnp.swapaxes(W_E, 0, 1)
    idx_t = jnp.swapaxes(x, 0, 1).astype(jnp.int32)
    out_t = _gather(table, idx_t, b, p)
    return jnp.transpose(out_t, (1, 0, 2))
